# baseline (speedup 1.0000x reference)
import functools

import jax
import jax.numpy as jnp
from jax import lax
from jax.experimental import pallas as pl
from jax.experimental.pallas import tpu as pltpu
from jax.experimental.pallas import tpu_sc as plsc

_B = 1024
_D_IN = 2048
_D_HID = 4096
_D_OUT = 2048
_N_HEADS = 8
_N_CLASSES = 1000
_NPAD = 1024
_BLK = 128
_NBLK = 16
_P = _NBLK * _BLK

_NC, _NS = 2, 16
_NW = _NC * _NS

_HI = jax.lax.Precision.HIGHEST


def _routing_body(t_col_ref, dest_ref, src_ref, meta_ref):
    f32 = jnp.float32
    t_col = t_col_ref[...]
    lane = lax.broadcasted_iota(jnp.int32, (_B, 128), 1)
    oh = (t_col == lane).astype(f32)
    row_i = lax.broadcasted_iota(jnp.int32, (_B, _B), 0)
    col_i = lax.broadcasted_iota(jnp.int32, (_B, _B), 1)
    strict_l = (col_i < row_i).astype(f32)
    cum = lax.dot(strict_l, oh, precision=_HI)
    counts = jnp.sum(oh, axis=0, keepdims=True)
    nblk = (counts.astype(jnp.int32) + (_BLK - 1)) // _BLK
    nblk_f = nblk.astype(f32)
    hrow = lax.broadcasted_iota(jnp.int32, (128, 128), 0)
    hcol = lax.broadcasted_iota(jnp.int32, (128, 128), 1)
    strict_u = (hrow < hcol).astype(f32)
    excl_blk = lax.dot(nblk_f, strict_u, precision=_HI)
    pad_off = excl_blk * float(_BLK)
    blk_end = excl_blk + nblk_f
    totblk = jnp.sum(nblk_f)
    dest_f = jnp.sum(oh * (cum + pad_off), axis=1, keepdims=True)
    dest_ref[...] = dest_f.astype(jnp.int32)
    pcol = lax.broadcasted_iota(jnp.int32, (_B, _P), 1)
    dmat = (dest_f.astype(jnp.int32) == pcol).astype(f32)
    tok_row = lax.broadcasted_iota(jnp.int32, (1, _B), 1).astype(f32)
    src_ref[...] = lax.dot(tok_row, dmat, precision=_HI).astype(jnp.int32)
    b_col = lax.broadcasted_iota(jnp.int32, (128, 128), 0).astype(f32)
    hmask = (hcol < _N_HEADS).astype(f32)
    cmp = jnp.where(b_col >= blk_end, 1.0, 0.0) * hmask
    bh_col = jnp.minimum(jnp.sum(cmp, axis=1, keepdims=True), 7.0)
    rowi = lax.broadcasted_iota(jnp.int32, (128, 1), 0)
    meta_ref[...] = jnp.where(rowi == _NBLK, totblk, bh_col).astype(jnp.int32)


def _route(task_ids):
    t_col = task_ids.astype(jnp.int32).reshape(_B, 1)
    dest, src, meta = pl.pallas_call(
        _routing_body,
        out_shape=[
            jax.ShapeDtypeStruct((_B, 1), jnp.int32),
            jax.ShapeDtypeStruct((1, _P), jnp.int32),
            jax.ShapeDtypeStruct((128, 1), jnp.int32),
        ],
    )(t_col)
    return dest.reshape(_B), src.reshape(_P), meta.reshape(128)


def _mlp1_body(x_ref, w_ref, b_ref, o_ref):
    acc = jnp.dot(x_ref[...], w_ref[...], preferred_element_type=jnp.float32)
    o_ref[...] = jnp.maximum(acc + b_ref[...], 0.0)


def _mlp2_body(h_ref, w_ref, b_ref, o_ref):
    acc = jnp.dot(h_ref[...], w_ref[...], preferred_element_type=jnp.float32)
    o_ref[...] = acc + b_ref[...]


def _base_mlp(x, W1, b1, W2, b2):
    bn = 512
    hid = pl.pallas_call(
        _mlp1_body,
        grid=(_D_HID // bn,),
        in_specs=[
            pl.BlockSpec((_B, _D_IN), lambda n: (0, 0)),
            pl.BlockSpec((_D_IN, bn), lambda n: (0, n)),
            pl.BlockSpec((1, bn), lambda n: (0, n)),
        ],
        out_specs=pl.BlockSpec((_B, bn), lambda n: (0, n)),
        out_shape=jax.ShapeDtypeStruct((_B, _D_HID), jnp.float32),
    )(x, W1, b1.reshape(1, _D_HID))
    feats = pl.pallas_call(
        _mlp2_body,
        grid=(_D_OUT // bn,),
        in_specs=[
            pl.BlockSpec((_B, _D_HID), lambda n: (0, 0)),
            pl.BlockSpec((_D_HID, bn), lambda n: (0, n)),
            pl.BlockSpec((1, bn), lambda n: (0, n)),
        ],
        out_specs=pl.BlockSpec((_B, bn), lambda n: (0, n)),
        out_shape=jax.ShapeDtypeStruct((_B, _D_OUT), jnp.float32),
    )(hid, W2, b2.reshape(1, _D_OUT))
    return feats


def _head_body(bh_ref, tb_ref, f_ref, w_ref, b_ref, o_ref):
    b = pl.program_id(0)

    @pl.when(b < tb_ref[0])
    def _():
        o_ref[...] = jnp.dot(f_ref[...], w_ref[0],
                             preferred_element_type=jnp.float32) + b_ref[0]

    @pl.when(b >= tb_ref[0])
    def _():
        o_ref[...] = jnp.zeros_like(o_ref)


def _heads(feats_sorted, whp, bhp, block_head, totblk):
    bn = 512
    grid_spec = pltpu.PrefetchScalarGridSpec(
        num_scalar_prefetch=2,
        grid=(_NBLK, _NPAD // bn),
        in_specs=[
            pl.BlockSpec((_BLK, _D_OUT), lambda b, n, bh, tb: (b, 0)),
            pl.BlockSpec((1, _D_OUT, bn), lambda b, n, bh, tb: (bh[b], 0, n)),
            pl.BlockSpec((1, 1, bn), lambda b, n, bh, tb: (bh[b], 0, n)),
        ],
        out_specs=pl.BlockSpec((_BLK, bn), lambda b, n, bh, tb: (b, n)),
    )
    return pl.pallas_call(
        _head_body,
        grid_spec=grid_spec,
        out_shape=jax.ShapeDtypeStruct((_P, _NPAD), jnp.float32),
    )(block_head, totblk, feats_sorted, whp, bhp)


def _sc_gather_rows(table, idx, chunk):
    bout = idx.shape[0]
    d = table.shape[1]
    b_per_w = bout // _NW
    nchunks = b_per_w // chunk
    mesh = plsc.VectorSubcoreMesh(core_axis_name="c", subcore_axis_name="s")

    @functools.partial(
        pl.kernel,
        mesh=mesh,
        out_type=jax.ShapeDtypeStruct((bout, d), jnp.float32),
        scratch_types=[
            pltpu.VMEM((chunk,), jnp.int32),
            pltpu.VMEM((chunk, d), jnp.float32),
            pltpu.SemaphoreType.DMA,
        ],
    )
    def k(table_hbm, idx_hbm, out_hbm, idx_v, rows_v, sem):
        wid = lax.axis_index("s") * _NC + lax.axis_index("c")
        for ci in range(nchunks):
            base = wid * b_per_w + ci * chunk
            pltpu.sync_copy(idx_hbm.at[pl.ds(base, chunk)], idx_v)
            pltpu.async_copy(table_hbm.at[idx_v], rows_v, sem).wait()
            pltpu.sync_copy(rows_v, out_hbm.at[pl.ds(base, chunk)])

    return k(table, idx)


def kernel(x, task_ids, W1, b1, W2, b2, Wh, bh):
    dest, src, meta = _route(task_ids)
    block_head = meta[:_NBLK]
    totblk = meta[_NBLK:_NBLK + 1]
    feats = _base_mlp(x, W1, b1, W2, b2)
    feats_sorted = _sc_gather_rows(feats, src, 32)
    whp = jnp.pad(Wh, ((0, 0), (0, 0), (0, _NPAD - _N_CLASSES)))
    bhp = jnp.pad(bh, ((0, 0), (0, _NPAD - _N_CLASSES))).reshape(_N_HEADS, 1, _NPAD)
    headout = _heads(feats_sorted, whp, bhp, block_head, totblk)
    out_rows = _sc_gather_rows(headout, dest, 32)
    return out_rows[:, :_N_CLASSES]


# speedup vs baseline: 1.3700x; 1.3700x over previous
import functools

import jax
import jax.numpy as jnp
from jax import lax
from jax.experimental import pallas as pl
from jax.experimental.pallas import tpu as pltpu
from jax.experimental.pallas import tpu_sc as plsc

_B = 1024
_D_IN = 2048
_D_HID = 4096
_D_OUT = 2048
_N_HEADS = 8
_N_CLASSES = 1000
_BLK = 128
_CAP = 120
_NBLK = 16
_P = _NBLK * _BLK

_NC, _NS = 2, 16
_NW = _NC * _NS

_HI = jax.lax.Precision.HIGHEST


def _routing_body(t_col_ref, t_row_ref, pi_ref, dest_ref, meta_ref):
    f32 = jnp.float32
    i32 = jnp.int32
    t_col = t_col_ref[...]
    t_row = t_row_ref[...]
    lane128 = lax.broadcasted_iota(i32, (_B, 128), 1)
    oh = (t_col == lane128).astype(f32)
    row_b = lax.broadcasted_iota(i32, (_B, _B), 0)
    col_b = lax.broadcasted_iota(i32, (_B, _B), 1)
    strict_l_b = (col_b < row_b).astype(f32)
    cum = lax.dot(strict_l_b, oh, precision=_HI)
    counts = jnp.sum(oh, axis=0, keepdims=True)
    r128 = lax.broadcasted_iota(i32, (128, 128), 0)
    c128 = lax.broadcasted_iota(i32, (128, 128), 1)
    strict_u = (r128 < c128).astype(f32)
    cexcl = lax.dot(counts, strict_u, precision=_HI)
    nblk_i = (counts.astype(i32) + (_CAP - 1)) // _CAP
    nblk = nblk_i.astype(f32)
    bexcl = lax.dot(nblk, strict_u, precision=_HI)
    totblk = jnp.sum(nblk)
    rank = jnp.sum(oh * cum, axis=1, keepdims=True)
    s_col = jnp.sum(oh * (cum + cexcl), axis=1, keepdims=True)
    pcol_b = lax.broadcasted_iota(i32, (_B, _B), 1)
    smat = (s_col.astype(i32) == pcol_b).astype(f32)
    tok_row = lax.broadcasted_iota(i32, (1, _B), 1).astype(f32)
    pi_ref[...] = lax.dot(tok_row, smat, precision=_HI).astype(i32)
    ohT = (lax.broadcasted_iota(i32, (128, _B), 0) == t_row).astype(f32)
    counts_col = lax.dot(ohT, jnp.ones((_B, 1), f32), precision=_HI)
    strict_l128 = (c128 < r128).astype(f32)
    cexcl_col = lax.dot(strict_l128, counts_col, precision=_HI)
    nblk_col = ((counts_col.astype(i32) + (_CAP - 1)) // _CAP).astype(f32)
    bexcl_col = lax.dot(strict_l128, nblk_col, precision=_HI)
    blk_end_col2 = bexcl_col + nblk_col
    hmask_col = (lax.broadcasted_iota(i32, (128, 1), 0) < _N_HEADS).astype(f32)
    M = jnp.where(c128.astype(f32) >= blk_end_col2, 1.0, 0.0) * hmask_col
    bh_row = jnp.minimum(lax.dot(jnp.ones((1, 128), f32), M, precision=_HI),
                         7.0)
    ohb2 = (lax.broadcasted_iota(i32, (128, 128), 0)
            == bh_row.astype(i32)).astype(f32)
    cexcl_by_b = lax.dot(cexcl, ohb2, precision=_HI)
    bexcl_by_b = lax.dot(bexcl, ohb2, precision=_HI)
    brow = lax.broadcasted_iota(i32, (1, 128), 1).astype(f32)
    seg_raw = cexcl_by_b + (brow - bexcl_by_b) * float(_CAP)
    seg_al = jnp.floor(seg_raw / 8.0) * 8.0
    seg_row = jnp.clip(seg_al, 0.0, float(_B - _BLK))
    bexcl_t = jnp.sum(oh * bexcl, axis=1, keepdims=True)
    b_i = bexcl_t + jnp.floor(rank / float(_CAP))
    ohbi = (b_i.astype(i32) == lane128).astype(f32)
    seg_t = jnp.sum(ohbi * seg_row, axis=1, keepdims=True)
    dest_ref[...] = (b_i * float(_BLK) + s_col - seg_t).astype(i32)
    lane_row = lax.broadcasted_iota(i32, (1, 128), 1)
    meta0 = jnp.where(lane_row == _NBLK, totblk, bh_row)
    meta_ref[...] = jnp.concatenate(
        [meta0, seg_row], axis=0).astype(i32)


def _route(task_ids):
    t = task_ids.astype(jnp.int32)
    pi, dest, meta = pl.pallas_call(
        _routing_body,
        out_shape=[
            jax.ShapeDtypeStruct((1, _B), jnp.int32),
            jax.ShapeDtypeStruct((_B, 1), jnp.int32),
            jax.ShapeDtypeStruct((2, 128), jnp.int32),
        ],
    )(t.reshape(_B, 1), t.reshape(1, _B))
    return pi.reshape(_B), dest.reshape(_B), meta


def _mlp1_body(x_ref, w_ref, b_ref, o_ref):
    acc = jnp.dot(x_ref[...], w_ref[...], preferred_element_type=jnp.float32)
    o_ref[...] = jnp.maximum(acc + b_ref[...], 0.0)


def _mlp2_body(h_ref, w_ref, b_ref, o_ref):
    acc = jnp.dot(h_ref[...], w_ref[...], preferred_element_type=jnp.float32)
    o_ref[...] = acc + b_ref[...]


def _base_mlp(x, W1, b1, W2, b2):
    bn = 512
    hid = pl.pallas_call(
        _mlp1_body,
        grid=(_D_HID // bn,),
        in_specs=[
            pl.BlockSpec((_B, _D_IN), lambda n: (0, 0)),
            pl.BlockSpec((_D_IN, bn), lambda n: (0, n)),
            pl.BlockSpec((1, bn), lambda n: (0, n)),
        ],
        out_specs=pl.BlockSpec((_B, bn), lambda n: (0, n)),
        out_shape=jax.ShapeDtypeStruct((_B, _D_HID), jnp.float32),
    )(x, W1, b1.reshape(1, _D_HID))
    feats = pl.pallas_call(
        _mlp2_body,
        grid=(_D_OUT // bn,),
        in_specs=[
            pl.BlockSpec((_B, _D_HID), lambda n: (0, 0)),
            pl.BlockSpec((_D_HID, bn), lambda n: (0, n)),
            pl.BlockSpec((1, bn), lambda n: (0, n)),
        ],
        out_specs=pl.BlockSpec((_B, bn), lambda n: (0, n)),
        out_shape=jax.ShapeDtypeStruct((_B, bn * (_D_OUT // bn)), jnp.float32),
    )(hid, W2, b2.reshape(1, _D_OUT))
    return feats


def _head_body(bh_ref, tb_ref, ss_ref, f_ref, w_ref, b_ref, o_ref):
    b = pl.program_id(0)

    @pl.when(b < tb_ref[0])
    def _():
        f = f_ref[pl.ds(pl.multiple_of(ss_ref[b], 8), _BLK), :]
        o_ref[:, :_N_CLASSES] = jnp.dot(
            f, w_ref[0], preferred_element_type=jnp.float32) + b_ref[0]


def _heads(feats_sorted, Wh, bh2, block_head, totblk, seg_start):
    grid_spec = pltpu.PrefetchScalarGridSpec(
        num_scalar_prefetch=3,
        grid=(_NBLK,),
        in_specs=[
            pl.BlockSpec((_B, _D_OUT), lambda b, bhi, tb, ss: (0, 0)),
            pl.BlockSpec((1, _D_OUT, _N_CLASSES), lambda b, bhi, tb, ss: (bhi[b], 0, 0)),
            pl.BlockSpec((1, 1, _N_CLASSES), lambda b, bhi, tb, ss: (bhi[b], 0, 0)),
        ],
        out_specs=pl.BlockSpec((_BLK, 1024), lambda b, bhi, tb, ss: (b, 0)),
    )
    return pl.pallas_call(
        _head_body,
        grid_spec=grid_spec,
        out_shape=jax.ShapeDtypeStruct((_P, 1024), jnp.float32),
    )(block_head, totblk, seg_start, feats_sorted, Wh, bh2)


def _sc_gather_rows(table, idx):
    bout = idx.shape[0]
    d = table.shape[1]
    b_per_w = bout // _NW
    mesh = plsc.VectorSubcoreMesh(core_axis_name="c", subcore_axis_name="s")

    @functools.partial(
        pl.kernel,
        mesh=mesh,
        out_type=jax.ShapeDtypeStruct((bout, d), table.dtype),
        scratch_types=[
            pltpu.VMEM((b_per_w,), jnp.int32),
            pltpu.VMEM((b_per_w, d), table.dtype),
            pltpu.SemaphoreType.DMA,
        ],
    )
    def k(table_hbm, idx_hbm, out_hbm, idx_v, rows_v, sem):
        wid = lax.axis_index("s") * _NC + lax.axis_index("c")
        base = wid * b_per_w
        pltpu.sync_copy(idx_hbm.at[pl.ds(base, b_per_w)], idx_v)
        pltpu.async_copy(table_hbm.at[idx_v], rows_v, sem).wait()
        pltpu.sync_copy(rows_v, out_hbm.at[pl.ds(base, b_per_w)])

    return k(table, idx)


def kernel(x, task_ids, W1, b1, W2, b2, Wh, bh):
    pi, dest, meta = _route(task_ids)
    block_head = meta[0, :_NBLK]
    totblk = meta[0, _NBLK:_NBLK + 1]
    seg_start = meta[1, :_NBLK]
    x_sorted = _sc_gather_rows(x, pi)
    feats = _base_mlp(x_sorted, W1, b1, W2, b2)
    bh2 = bh.reshape(_N_HEADS, 1, _N_CLASSES)
    headout = _heads(feats, Wh, bh2, block_head, totblk, seg_start)
    return _sc_gather_rows(headout, dest)[:, :_N_CLASSES]


# speedup vs baseline: 1.8910x; 1.3803x over previous
import functools

import jax
import jax.numpy as jnp
from jax import lax
from jax.experimental import pallas as pl
from jax.experimental.pallas import tpu as pltpu
from jax.experimental.pallas import tpu_sc as plsc

_B = 1024
_D_IN = 2048
_D_HID = 4096
_D_OUT = 2048
_N_HEADS = 8
_N_CLASSES = 1000
_BLK = 128
_CAP = 120
_NBLK = 16
_P = _NBLK * _BLK

_NC, _NS = 2, 16
_NW = _NC * _NS

_HI = jax.lax.Precision.HIGHEST


def _routing_body(t_col_ref, t_row_ref, pi_ref, dest_ref, meta_ref):
    f32 = jnp.float32
    i32 = jnp.int32
    t_col = t_col_ref[...]
    t_row = t_row_ref[...]
    lane128 = lax.broadcasted_iota(i32, (_B, 128), 1)
    oh = (t_col == lane128).astype(f32)
    row_b = lax.broadcasted_iota(i32, (_B, _B), 0)
    col_b = lax.broadcasted_iota(i32, (_B, _B), 1)
    strict_l_b = (col_b < row_b).astype(f32)
    cum = lax.dot(strict_l_b, oh, precision=_HI)
    counts = jnp.sum(oh, axis=0, keepdims=True)
    r128 = lax.broadcasted_iota(i32, (128, 128), 0)
    c128 = lax.broadcasted_iota(i32, (128, 128), 1)
    strict_u = (r128 < c128).astype(f32)
    cexcl = lax.dot(counts, strict_u, precision=_HI)
    nblk_i = (counts.astype(i32) + (_CAP - 1)) // _CAP
    nblk = nblk_i.astype(f32)
    bexcl = lax.dot(nblk, strict_u, precision=_HI)
    totblk = jnp.sum(nblk)
    rank = jnp.sum(oh * cum, axis=1, keepdims=True)
    s_col = jnp.sum(oh * (cum + cexcl), axis=1, keepdims=True)
    pcol_b = lax.broadcasted_iota(i32, (_B, _B), 1)
    smat = (s_col.astype(i32) == pcol_b).astype(f32)
    tok_row = lax.broadcasted_iota(i32, (1, _B), 1).astype(f32)
    pi_ref[...] = lax.dot(tok_row, smat, precision=_HI).astype(i32)
    ohT = (lax.broadcasted_iota(i32, (128, _B), 0) == t_row).astype(f32)
    counts_col = lax.dot(ohT, jnp.ones((_B, 1), f32), precision=_HI)
    strict_l128 = (c128 < r128).astype(f32)
    cexcl_col = lax.dot(strict_l128, counts_col, precision=_HI)
    nblk_col = ((counts_col.astype(i32) + (_CAP - 1)) // _CAP).astype(f32)
    bexcl_col = lax.dot(strict_l128, nblk_col, precision=_HI)
    blk_end_col2 = bexcl_col + nblk_col
    hmask_col = (lax.broadcasted_iota(i32, (128, 1), 0) < _N_HEADS).astype(f32)
    M = jnp.where(c128.astype(f32) >= blk_end_col2, 1.0, 0.0) * hmask_col
    bh_row = jnp.minimum(lax.dot(jnp.ones((1, 128), f32), M, precision=_HI),
                         7.0)
    ohb2 = (lax.broadcasted_iota(i32, (128, 128), 0)
            == bh_row.astype(i32)).astype(f32)
    cexcl_by_b = lax.dot(cexcl, ohb2, precision=_HI)
    bexcl_by_b = lax.dot(bexcl, ohb2, precision=_HI)
    brow = lax.broadcasted_iota(i32, (1, 128), 1).astype(f32)
    seg_raw = cexcl_by_b + (brow - bexcl_by_b) * float(_CAP)
    seg_al = jnp.floor(seg_raw / 8.0) * 8.0
    seg_row = jnp.clip(seg_al, 0.0, float(_B - _BLK))
    bexcl_t = jnp.sum(oh * bexcl, axis=1, keepdims=True)
    b_i = bexcl_t + jnp.floor(rank / float(_CAP))
    ohbi = (b_i.astype(i32) == lane128).astype(f32)
    seg_t = jnp.sum(ohbi * seg_row, axis=1, keepdims=True)
    dest_ref[...] = (b_i * float(_BLK) + s_col - seg_t).astype(i32)
    lane_row = lax.broadcasted_iota(i32, (1, 128), 1)
    meta0 = jnp.where(lane_row == _NBLK, totblk, bh_row)
    meta_ref[...] = jnp.concatenate(
        [meta0, seg_row], axis=0).astype(i32)


def _route(task_ids):
    t = task_ids.astype(jnp.int32)
    pi, dest, meta = pl.pallas_call(
        _routing_body,
        out_shape=[
            jax.ShapeDtypeStruct((1, _B), jnp.int32),
            jax.ShapeDtypeStruct((_B, 1), jnp.int32),
            jax.ShapeDtypeStruct((2, 128), jnp.int32),
        ],
    )(t.reshape(_B, 1), t.reshape(1, _B))
    return pi.reshape(_B), dest.reshape(_B), meta


def _mlp1_body(x_ref, w_ref, b_ref, o_ref):
    acc = jnp.dot(x_ref[...], w_ref[...], preferred_element_type=jnp.float32)
    o_ref[...] = jnp.maximum(acc + b_ref[...], 0.0)


def _mlp2_body(h_ref, w_ref, b_ref, o_ref):
    acc = jnp.dot(h_ref[...], w_ref[...], preferred_element_type=jnp.float32)
    o_ref[...] = acc + b_ref[...]


def _base_mlp(x, W1, b1, W2, b2):
    bn = 512
    hid = pl.pallas_call(
        _mlp1_body,
        grid=(_D_HID // bn,),
        in_specs=[
            pl.BlockSpec((_B, _D_IN), lambda n: (0, 0)),
            pl.BlockSpec((_D_IN, bn), lambda n: (0, n)),
            pl.BlockSpec((1, bn), lambda n: (0, n)),
        ],
        out_specs=pl.BlockSpec((_B, bn), lambda n: (0, n)),
        out_shape=jax.ShapeDtypeStruct((_B, _D_HID), jnp.float32),
    )(x, W1, b1.reshape(1, _D_HID))
    feats = pl.pallas_call(
        _mlp2_body,
        grid=(_D_OUT // bn,),
        in_specs=[
            pl.BlockSpec((_B, _D_HID), lambda n: (0, 0)),
            pl.BlockSpec((_D_HID, bn), lambda n: (0, n)),
            pl.BlockSpec((1, bn), lambda n: (0, n)),
        ],
        out_specs=pl.BlockSpec((_B, bn), lambda n: (0, n)),
        out_shape=jax.ShapeDtypeStruct((_B, bn * (_D_OUT // bn)), jnp.float32),
    )(hid, W2, b2.reshape(1, _D_OUT))
    return feats


def _head_body(bh_ref, tb_ref, ss_ref, f_ref, w_ref, b_ref, o_ref):
    b = pl.program_id(0)

    @pl.when(b < tb_ref[0])
    def _():
        f = f_ref[pl.ds(pl.multiple_of(ss_ref[b], 8), _BLK), :]
        acc = lax.dot_general(f, w_ref[0], (((1,), (1,)), ((), ())),
                              preferred_element_type=jnp.float32)
        o_ref[:, :_N_CLASSES] = acc + b_ref[0]


def _heads(feats_sorted, WhT, bh2, block_head, totblk, seg_start):
    grid_spec = pltpu.PrefetchScalarGridSpec(
        num_scalar_prefetch=3,
        grid=(_NBLK,),
        in_specs=[
            pl.BlockSpec((_B, _D_OUT), lambda b, bhi, tb, ss: (0, 0)),
            pl.BlockSpec((1, _N_CLASSES, _D_OUT), lambda b, bhi, tb, ss: (bhi[b], 0, 0)),
            pl.BlockSpec((1, 1, _N_CLASSES), lambda b, bhi, tb, ss: (bhi[b], 0, 0)),
        ],
        out_specs=pl.BlockSpec((_BLK, 1024), lambda b, bhi, tb, ss: (b, 0)),
    )
    return pl.pallas_call(
        _head_body,
        grid_spec=grid_spec,
        out_shape=jax.ShapeDtypeStruct((_P, 1024), jnp.float32),
    )(block_head, totblk, seg_start, feats_sorted, WhT, bh2)


def _sc_gather_rows(table, idx):
    bout = idx.shape[0]
    d = table.shape[1]
    b_per_w = bout // _NW
    mesh = plsc.VectorSubcoreMesh(core_axis_name="c", subcore_axis_name="s")

    @functools.partial(
        pl.kernel,
        mesh=mesh,
        out_type=jax.ShapeDtypeStruct((bout, d), table.dtype),
        scratch_types=[
            pltpu.VMEM((b_per_w,), jnp.int32),
            pltpu.VMEM((b_per_w, d), table.dtype),
            pltpu.SemaphoreType.DMA,
        ],
    )
    def k(table_hbm, idx_hbm, out_hbm, idx_v, rows_v, sem):
        wid = lax.axis_index("s") * _NC + lax.axis_index("c")
        base = wid * b_per_w
        pltpu.sync_copy(idx_hbm.at[pl.ds(base, b_per_w)], idx_v)
        pltpu.async_copy(table_hbm.at[idx_v], rows_v, sem).wait()
        pltpu.sync_copy(rows_v, out_hbm.at[pl.ds(base, b_per_w)])

    return k(table, idx)


def kernel(x, task_ids, W1, b1, W2, b2, Wh, bh):
    pi, dest, meta = _route(task_ids)
    block_head = meta[0, :_NBLK]
    totblk = meta[0, _NBLK:_NBLK + 1]
    seg_start = meta[1, :_NBLK]
    x_sorted = _sc_gather_rows(x, pi)
    feats = _base_mlp(x_sorted, W1, b1, W2, b2)
    bh2 = bh.reshape(_N_HEADS, 1, _N_CLASSES)
    wht = jnp.transpose(Wh, (0, 2, 1))
    headout = _heads(feats, wht, bh2, block_head, totblk, seg_start)
    return _sc_gather_rows(headout, dest)[:, :_N_CLASSES]
